# Initial kernel scaffold; baseline (speedup 1.0000x reference)
import jax
import jax.numpy as jnp
from jax.experimental import pallas as pl


def kernel(rgb, K, feat, params):
    raise NotImplementedError("write your pallas kernel here")


# speedup vs baseline: 4.5720x; 4.5720x over previous
import jax
import jax.numpy as jnp
from jax.experimental import pallas as pl
from jax.experimental.pallas import tpu as pltpu

B = 2
H, W = 352, 1216
FH, FW = 88, 304
EMBED = 64
REFINE = 48
DBINS = 128
OUT = 4
GX, GY, GZ = 128, 32, 128
MPV = 0.2
OFF = (-12.8, -1.0, 0.0)


def _conv2d(x, w, b=None, pad=0):
    y = jax.lax.conv_general_dilated(x, w, (1, 1), [(pad, pad)] * 2,
                                     dimension_numbers=('NCHW', 'OIHW', 'NCHW'))
    return y if b is None else y + b[None, :, None, None]


def _bn(x, g, b):
    m = x.mean((0, 2, 3), keepdims=True)
    v = x.var((0, 2, 3), keepdims=True)
    return (x - m) / jnp.sqrt(v + 1e-5) * g[None, :, None, None] + b[None, :, None, None]


def _block(x, p):
    h = jax.nn.relu(_bn(_conv2d(x, p['w1'], pad=1), p['g1'], p['b1']))
    h = _bn(_conv2d(h, p['w2'], pad=1), p['g2'], p['b2'])
    return jax.nn.relu(h + x)


def _resize_bilinear_ac(x, oh, ow):
    Bx, C, Hx, Wx = x.shape
    ys = jnp.linspace(0., Hx - 1., oh)
    xs = jnp.linspace(0., Wx - 1., ow)
    y0 = jnp.floor(ys).astype(jnp.int32); y1 = jnp.minimum(y0 + 1, Hx - 1)
    x0 = jnp.floor(xs).astype(jnp.int32); x1 = jnp.minimum(x0 + 1, Wx - 1)
    wy = (ys - y0)[None, None, :, None]
    wx = (xs - x0)[None, None, None, :]
    row = x[:, :, y0, :] * (1. - wy) + x[:, :, y1, :] * wy
    return row[:, :, :, x0] * (1. - wx) + row[:, :, :, x1] * wx


def _splat_body(bias_ref, t_ref, pz_ref, cu_ref, rv_ref, out_ref):
    T = t_ref[0]
    pz = pz_ref[0, 0]
    cu = cu_ref[0, 0]
    rv = rv_ref[0, 0]
    f32 = jnp.float32
    s = jax.lax.dot_general(T, rv, (((0,), (1,)), ((), ())),
                            preferred_element_type=f32)
    t1b = jax.lax.dot_general(pz, rv, (((0,), (1,)), ((), ())),
                              preferred_element_type=f32)
    wgt = jax.lax.dot_general(t1b, cu, (((0,), (0,)), ((), ())),
                              preferred_element_type=f32)
    for c in range(REFINE):
        sc = s[c * FW:(c + 1) * FW, :]
        oc = jax.lax.dot_general(sc, cu, (((0,), (0,)), ((), ())),
                                 preferred_element_type=f32)
        out_ref[0, 0, c] = oc * wgt + bias_ref[c]


def _lift_splat_pallas(rctx, probz, cu, rv, rbias):
    T2 = rctx.transpose(0, 2, 1, 3).reshape(B, FH, REFINE * FW)
    grid = (B, GZ)
    return pl.pallas_call(
        _splat_body,
        out_shape=jax.ShapeDtypeStruct((B, GZ, REFINE, GY, GX), jnp.float32),
        grid=grid,
        in_specs=[
            pl.BlockSpec(memory_space=pltpu.SMEM),
            pl.BlockSpec((1, FH, REFINE * FW), lambda b, z: (b, 0, 0)),
            pl.BlockSpec((1, 1, FH, FW), lambda b, z: (b, z, 0, 0)),
            pl.BlockSpec((1, 1, FW, GX), lambda b, z: (b, z, 0, 0)),
            pl.BlockSpec((1, 1, GY, FH), lambda b, z: (b, z, 0, 0)),
        ],
        out_specs=pl.BlockSpec((1, 1, REFINE, GY, GX),
                               lambda b, z: (b, z, 0, 0, 0)),
        compiler_params=pltpu.CompilerParams(
            dimension_semantics=("parallel", "arbitrary"),
        ),
        name="lift_splat",
    )(rbias, T2, probz, cu, rv)


def kernel(rgb, K, feat, params):
    f32 = jnp.float32
    dp = params['dproj']
    x = jax.nn.relu(_conv2d(feat, dp['w0'], dp['b0']))
    x = _block(x, dp['bb1'])
    x = _block(x, dp['bb2'])
    depth = _conv2d(x, dp['w1'], dp['b1'])
    rgb_r = _resize_bilinear_ac(rgb, FH, FW)
    cp = params['cproj']
    c = jax.nn.relu(_conv2d(jnp.concatenate([rgb_r, feat], axis=1), cp['w0'], cp['b0']))
    c = _block(c, cp['bb1'])
    c = _block(c, cp['bb2'])
    context = _conv2d(c, cp['w1'], cp['b1'])

    prob = jax.nn.softmax(depth, axis=1)
    rctx = jnp.einsum('bchw,oc->bohw', context, params['rproj']['w'])

    ixs = jnp.arange(GX); iys = jnp.arange(GY); izs = jnp.arange(GZ)
    Xc = ((ixs + 0.5) * MPV + OFF[0]).astype(f32)
    Yc = ((iys + 0.5) * MPV + OFF[1]).astype(f32)
    Zc = ((izs + 0.5) * MPV + OFF[2]).astype(f32)
    fx = K[:, 0, 0][:, None, None]; fy = K[:, 1, 1][:, None, None]
    cx = K[:, 0, 2][:, None, None]; cy = K[:, 1, 2][:, None, None]
    Zs = jnp.maximum(Zc, 1e-6)[None, None, :]
    u = fx * Xc[None, :, None] / Zs + cx
    v = fy * Yc[None, :, None] / Zs + cy
    uval = (u >= 0) & (u < W)
    vval = (v >= 0) & (v < H)
    ui = jnp.clip(jnp.floor(u * (FW / W)), 0, FW - 1).astype(jnp.int32)
    vi = jnp.clip(jnp.floor(v * (FH / H)), 0, FH - 1).astype(jnp.int32)
    dbin = jnp.floor(Zc / MPV).astype(jnp.int32)
    zval = (Zc > 1e-3) & (dbin >= 0) & (dbin < DBINS)
    dbin_c = jnp.clip(dbin, 0, DBINS - 1)

    uiT = ui.transpose(0, 2, 1); uvalT = uval.transpose(0, 2, 1)
    viT = vi.transpose(0, 2, 1); vvalT = vval.transpose(0, 2, 1)
    cu = ((jnp.arange(FW)[None, None, :, None] == uiT[:, :, None, :])
          & uvalT[:, :, None, :] & zval[None, :, None, None]).astype(f32)
    rv = ((viT[:, :, :, None] == jnp.arange(FH)[None, None, None, :])
          & vvalT[:, :, :, None]).astype(f32)
    probz = jnp.take(prob, dbin_c, axis=1)

    g5 = _lift_splat_pallas(rctx, probz, cu, rv, params['rproj']['b'])

    rb = params['rblok']
    r = jax.lax.conv_general_dilated(
        g5, rb['w'], (1, 1, 1), [(1, 1)] * 3,
        dimension_numbers=('NDCHW', 'OIWHD', 'NDCHW')) \
        + rb['b'][None, None, :, None, None]
    xr = r.reshape(B, GZ, 8, REFINE // 8, GY, GX)
    m = xr.mean((1, 3, 4, 5), keepdims=True)
    vv = xr.var((1, 3, 4, 5), keepdims=True)
    xn = ((xr - m) / jnp.sqrt(vv + 1e-5)).reshape(r.shape)
    r = xn * rb['g'][None, None, :, None, None] + rb['be'][None, None, :, None, None]
    g = jax.nn.relu(g5 + r)
    out = jnp.einsum('bzcyx,oc->bzoyx', g, params['rout']['w']) \
        + params['rout']['b'][None, None, :, None, None]
    return out.transpose(0, 2, 4, 3, 1)


# speedup vs baseline: 6.8530x; 1.4989x over previous
import jax
import jax.numpy as jnp
from jax.experimental import pallas as pl
from jax.experimental.pallas import tpu as pltpu

B = 2
H, W = 352, 1216
FH, FW = 88, 304
EMBED = 64
REFINE = 48
DBINS = 128
OUT = 4
GX, GY, GZ = 128, 32, 128
MPV = 0.2
OFF = (-12.8, -1.0, 0.0)


def _conv2d(x, w, b=None, pad=0):
    y = jax.lax.conv_general_dilated(x, w, (1, 1), [(pad, pad)] * 2,
                                     dimension_numbers=('NCHW', 'OIHW', 'NCHW'))
    return y if b is None else y + b[None, :, None, None]


def _bn(x, g, b):
    m = x.mean((0, 2, 3), keepdims=True)
    v = x.var((0, 2, 3), keepdims=True)
    return (x - m) / jnp.sqrt(v + 1e-5) * g[None, :, None, None] + b[None, :, None, None]


def _block(x, p):
    h = jax.nn.relu(_bn(_conv2d(x, p['w1'], pad=1), p['g1'], p['b1']))
    h = _bn(_conv2d(h, p['w2'], pad=1), p['g2'], p['b2'])
    return jax.nn.relu(h + x)


def _resize_bilinear_ac(x, oh, ow):
    Bx, C, Hx, Wx = x.shape
    ys = jnp.linspace(0., Hx - 1., oh)
    xs = jnp.linspace(0., Wx - 1., ow)
    y0 = jnp.floor(ys).astype(jnp.int32); y1 = jnp.minimum(y0 + 1, Hx - 1)
    x0 = jnp.floor(xs).astype(jnp.int32); x1 = jnp.minimum(x0 + 1, Wx - 1)
    wy = (ys - y0)[None, None, :, None]
    wx = (xs - x0)[None, None, None, :]
    row = x[:, :, y0, :] * (1. - wy) + x[:, :, y1, :] * wy
    return row[:, :, :, x0] * (1. - wx) + row[:, :, :, x1] * wx


def _splat_body(bias_ref, t_ref, pz_ref, cu_ref, rv_ref, out_ref):
    T = t_ref[0]
    pz = pz_ref[0, 0]
    cu = cu_ref[0, 0]
    rv = rv_ref[0, 0]
    f32 = jnp.float32
    s = jax.lax.dot_general(T, rv, (((0,), (1,)), ((), ())),
                            preferred_element_type=f32)
    t1b = jax.lax.dot_general(pz, rv, (((0,), (1,)), ((), ())),
                              preferred_element_type=f32)
    wgt = jax.lax.dot_general(t1b, cu, (((0,), (0,)), ((), ())),
                              preferred_element_type=f32)
    for c in range(REFINE):
        sc = s[c * FW:(c + 1) * FW, :]
        oc = jax.lax.dot_general(sc, cu, (((0,), (0,)), ((), ())),
                                 preferred_element_type=f32)
        out_ref[0, 0, c] = oc * wgt + bias_ref[c]


def _lift_splat_pallas(rctx, probz, cu, rv, rbias):
    T2 = rctx.transpose(0, 2, 1, 3).reshape(B, FH, REFINE * FW)
    grid = (B, GZ)
    return pl.pallas_call(
        _splat_body,
        out_shape=jax.ShapeDtypeStruct((B, GZ, REFINE, GY, GX), jnp.float32),
        grid=grid,
        in_specs=[
            pl.BlockSpec(memory_space=pltpu.SMEM),
            pl.BlockSpec((1, FH, REFINE * FW), lambda b, z: (b, 0, 0)),
            pl.BlockSpec((1, 1, FH, FW), lambda b, z: (b, z, 0, 0)),
            pl.BlockSpec((1, 1, FW, GX), lambda b, z: (b, z, 0, 0)),
            pl.BlockSpec((1, 1, GY, FH), lambda b, z: (b, z, 0, 0)),
        ],
        out_specs=pl.BlockSpec((1, 1, REFINE, GY, GX),
                               lambda b, z: (b, z, 0, 0, 0)),
        compiler_params=pltpu.CompilerParams(
            dimension_semantics=("parallel", "arbitrary"),
        ),
        name="lift_splat",
    )(rbias, T2, probz, cu, rv)


def _conv3d_body(w_ref, gm_ref, g0_ref, gp_ref, r_ref, st_ref):
    f32 = jnp.float32
    iz = pl.program_id(1)
    gm = gm_ref[0, 0] * (iz > 0).astype(f32)
    g0 = g0_ref[0, 0]
    gp = gp_ref[0, 0] * (iz < GZ - 1).astype(f32)
    planes = (gm, g0, gp)
    lane = jax.lax.broadcasted_iota(jnp.int32, (REFINE, GY * GX), 1)
    xpos = lane % GX
    acc = jnp.zeros((REFINE, GY * GX), f32)
    for ky in range(3):
        d = ky - 1
        for kx in range(3):
            e = kx - 1
            a = jnp.zeros((REFINE, GY * GX), f32)
            for kz in range(3):
                w = w_ref[(ky * 3 + kx) * 3 + kz]
                a = a + jax.lax.dot_general(
                    w, planes[kz], (((1,), (0,)), ((), ())),
                    preferred_element_type=f32)
            shift = d * GX + e
            if shift != 0:
                a = jnp.roll(a, -shift, axis=1)
            cond = None
            if d == 1:
                cond = lane < (GY - 1) * GX
            elif d == -1:
                cond = lane >= GX
            if e == 1:
                c2 = xpos < GX - 1
                cond = c2 if cond is None else (cond & c2)
            elif e == -1:
                c2 = xpos > 0
                cond = c2 if cond is None else (cond & c2)
            if cond is not None:
                a = jnp.where(cond, a, 0.)
            acc = acc + a
    r_ref[0, 0] = acc
    st_ref[0, 0, 0] = jnp.sum(acc, axis=1)
    st_ref[0, 0, 1] = jnp.sum(acc * acc, axis=1)


def _conv3d_pallas(g4, w):
    nzm1 = GZ - 1
    return pl.pallas_call(
        _conv3d_body,
        out_shape=(jax.ShapeDtypeStruct((B, GZ, REFINE, GY * GX), jnp.float32),
                   jax.ShapeDtypeStruct((B, GZ, 2, REFINE), jnp.float32)),
        grid=(B, GZ),
        in_specs=[
            pl.BlockSpec((27, REFINE, REFINE), lambda b, z: (0, 0, 0)),
            pl.BlockSpec((1, 1, REFINE, GY * GX),
                         lambda b, z: (b, jnp.maximum(z - 1, 0), 0, 0)),
            pl.BlockSpec((1, 1, REFINE, GY * GX), lambda b, z: (b, z, 0, 0)),
            pl.BlockSpec((1, 1, REFINE, GY * GX),
                         lambda b, z: (b, jnp.minimum(z + 1, nzm1), 0, 0)),
        ],
        out_specs=(pl.BlockSpec((1, 1, REFINE, GY * GX), lambda b, z: (b, z, 0, 0)),
                   pl.BlockSpec((1, 1, 2, REFINE), lambda b, z: (b, z, 0, 0))),
        compiler_params=pltpu.CompilerParams(
            dimension_semantics=("parallel", "arbitrary"),
        ),
        name="conv3d_gnstats",
    )(w, g4, g4, g4)


def _finish_body(aff_ref, wo_ref, g_ref, r_ref, out_ref):
    f32 = jnp.float32
    g = g_ref[0, 0]
    r = r_ref[0, 0]
    ones = jnp.ones((1, GY * GX), f32)
    r_aug = jnp.concatenate([r, ones], axis=0)
    aff = aff_ref[0]
    rn = jax.lax.dot_general(aff, r_aug, (((1,), (0,)), ((), ())),
                             preferred_element_type=f32)
    h = jnp.maximum(g + rn, 0.)
    h_aug = jnp.concatenate([h, ones], axis=0)
    wo = wo_ref[...]
    out_ref[0, 0] = jax.lax.dot_general(wo, h_aug, (((1,), (0,)), ((), ())),
                                        preferred_element_type=f32)


def _finish_pallas(g4, r4, aff, wo_aug):
    return pl.pallas_call(
        _finish_body,
        out_shape=jax.ShapeDtypeStruct((B, GZ, OUT, GY * GX), jnp.float32),
        grid=(B, GZ),
        in_specs=[
            pl.BlockSpec((1, REFINE, REFINE + 1), lambda b, z: (b, 0, 0)),
            pl.BlockSpec((OUT, REFINE + 1), lambda b, z: (0, 0)),
            pl.BlockSpec((1, 1, REFINE, GY * GX), lambda b, z: (b, z, 0, 0)),
            pl.BlockSpec((1, 1, REFINE, GY * GX), lambda b, z: (b, z, 0, 0)),
        ],
        out_specs=pl.BlockSpec((1, 1, OUT, GY * GX), lambda b, z: (b, z, 0, 0)),
        compiler_params=pltpu.CompilerParams(
            dimension_semantics=("parallel", "arbitrary"),
        ),
        name="gn_relu_out",
    )(aff, wo_aug, g4, r4)


def kernel(rgb, K, feat, params):
    f32 = jnp.float32
    dp = params['dproj']
    x = jax.nn.relu(_conv2d(feat, dp['w0'], dp['b0']))
    x = _block(x, dp['bb1'])
    x = _block(x, dp['bb2'])
    depth = _conv2d(x, dp['w1'], dp['b1'])
    rgb_r = _resize_bilinear_ac(rgb, FH, FW)
    cp = params['cproj']
    c = jax.nn.relu(_conv2d(jnp.concatenate([rgb_r, feat], axis=1), cp['w0'], cp['b0']))
    c = _block(c, cp['bb1'])
    c = _block(c, cp['bb2'])
    context = _conv2d(c, cp['w1'], cp['b1'])

    prob = jax.nn.softmax(depth, axis=1)
    rctx = jnp.einsum('bchw,oc->bohw', context, params['rproj']['w'])

    ixs = jnp.arange(GX); iys = jnp.arange(GY); izs = jnp.arange(GZ)
    Xc = ((ixs + 0.5) * MPV + OFF[0]).astype(f32)
    Yc = ((iys + 0.5) * MPV + OFF[1]).astype(f32)
    Zc = ((izs + 0.5) * MPV + OFF[2]).astype(f32)
    fx = K[:, 0, 0][:, None, None]; fy = K[:, 1, 1][:, None, None]
    cx = K[:, 0, 2][:, None, None]; cy = K[:, 1, 2][:, None, None]
    Zs = jnp.maximum(Zc, 1e-6)[None, None, :]
    u = fx * Xc[None, :, None] / Zs + cx
    v = fy * Yc[None, :, None] / Zs + cy
    uval = (u >= 0) & (u < W)
    vval = (v >= 0) & (v < H)
    ui = jnp.clip(jnp.floor(u * (FW / W)), 0, FW - 1).astype(jnp.int32)
    vi = jnp.clip(jnp.floor(v * (FH / H)), 0, FH - 1).astype(jnp.int32)
    dbin = jnp.floor(Zc / MPV).astype(jnp.int32)
    zval = (Zc > 1e-3) & (dbin >= 0) & (dbin < DBINS)
    dbin_c = jnp.clip(dbin, 0, DBINS - 1)

    uiT = ui.transpose(0, 2, 1); uvalT = uval.transpose(0, 2, 1)
    viT = vi.transpose(0, 2, 1); vvalT = vval.transpose(0, 2, 1)
    cu = ((jnp.arange(FW)[None, None, :, None] == uiT[:, :, None, :])
          & uvalT[:, :, None, :] & zval[None, :, None, None]).astype(f32)
    rv = ((viT[:, :, :, None] == jnp.arange(FH)[None, None, None, :])
          & vvalT[:, :, :, None]).astype(f32)
    probz = jnp.take(prob, dbin_c, axis=1)

    g5 = _lift_splat_pallas(rctx, probz, cu, rv, params['rproj']['b'])

    rb = params['rblok']
    g4 = g5.reshape(B, GZ, REFINE, GY * GX)
    wt = jnp.transpose(rb['w'], (3, 2, 4, 0, 1)).reshape(27, REFINE, REFINE)
    r4, st = _conv3d_pallas(g4, wt)
    ch_sum = st[:, :, 0, :].sum(1)
    ch_sq = st[:, :, 1, :].sum(1)
    bconv = rb['b']
    npix = GZ * GY * GX
    s1 = ch_sum + npix * bconv[None, :]
    s2 = ch_sq + 2. * bconv[None, :] * ch_sum + npix * bconv[None, :] ** 2
    cnt = npix * (REFINE // 8)
    mean = s1.reshape(B, 8, REFINE // 8).sum(2) / cnt
    var = s2.reshape(B, 8, REFINE // 8).sum(2) / cnt - mean ** 2
    mean_c = jnp.repeat(mean, REFINE // 8, axis=1)
    var_c = jnp.repeat(var, REFINE // 8, axis=1)
    scale = rb['g'][None, :] / jnp.sqrt(var_c + 1e-5)
    shift = rb['be'][None, :] + (bconv[None, :] - mean_c) * scale
    aff = jnp.concatenate(
        [scale[:, :, None] * jnp.eye(REFINE, dtype=f32)[None], shift[:, :, None]],
        axis=2)
    wo_aug = jnp.concatenate(
        [params['rout']['w'], params['rout']['b'][:, None]], axis=1)
    out4 = _finish_pallas(g4, r4, aff, wo_aug)
    return out4.reshape(B, GZ, OUT, GY, GX).transpose(0, 2, 4, 3, 1)
